# Initial kernel scaffold; baseline (speedup 1.0000x reference)
import jax
import jax.numpy as jnp
from jax.experimental import pallas as pl


def kernel(z, u, x, edge_index, batch, batch_size, prev_h, W_xz, b_xz, W_hz, b_hz, W_xr, b_xr, W_hr, b_hr, W_xh, b_xh, W_hh, b_hh, Wg, bg):
    raise NotImplementedError("write your pallas kernel here")


# speedup vs baseline: 2.3161x; 2.3161x over previous
import functools

import jax
import jax.numpy as jnp
from jax import lax
from jax.experimental import pallas as pl
from jax.experimental.pallas import tpu as pltpu
from jax.experimental.pallas import tpu_sc as plsc

_N = 10000
_B = 256
_DZ = 128
_DH = 256

_NC = 2
_NS = 16
_NW = _NC * _NS

_RPW = 320
_CHUNK = 80
_NCHUNKS = _RPW // _CHUNK


def _seg_sum_body(z_hbm, bidx_hbm, out_hbm, idx_v, rows_v, zeros_v, acc_sh):
    c = lax.axis_index("c")
    s = lax.axis_index("s")
    w = s * _NC + c

    for i in range(16):
        for j in range(_DZ // 16):
            zeros_v[i, pl.ds(j * 16, 16)] = jnp.zeros((16,), jnp.float32)
    pltpu.sync_copy(zeros_v, acc_sh.at[pl.ds(s * 16, 16)])
    plsc.subcore_barrier()

    pltpu.sync_copy(bidx_hbm.at[w], idx_v)

    @pl.when(w < _NW - 1)
    def _full():
        pltpu.sync_copy(z_hbm.at[pl.ds(w * _RPW, _RPW)], rows_v)
        for j in range(_NCHUNKS):
            pltpu.sync_copy(
                rows_v.at[pl.ds(j * _CHUNK, _CHUNK)],
                acc_sh.at[idx_v.at[j]],
                add=True,
            )

    @pl.when(w == _NW - 1)
    def _tail():
        pltpu.sync_copy(
            z_hbm.at[pl.ds((_NW - 1) * _RPW, _CHUNK)], rows_v.at[pl.ds(0, _CHUNK)]
        )
        pltpu.sync_copy(
            rows_v.at[pl.ds(0, _CHUNK)], acc_sh.at[idx_v.at[0]], add=True
        )

    plsc.subcore_barrier()
    pltpu.sync_copy(
        acc_sh.at[pl.ds(s * 16, 16)], out_hbm.at[c, pl.ds(s * 16, 16)]
    )


_seg_sum = pl.kernel(
    _seg_sum_body,
    out_type=jax.ShapeDtypeStruct((_NC, _B, _DZ), jnp.float32),
    mesh=plsc.VectorSubcoreMesh(core_axis_name="c", subcore_axis_name="s"),
    scratch_types=[
        pltpu.VMEM((_NCHUNKS, _CHUNK), jnp.int32),
        pltpu.VMEM((_RPW, _DZ), jnp.float32),
        pltpu.VMEM((16, _DZ), jnp.float32),
        pltpu.VMEM_SHARED((_B, _DZ), jnp.float32),
    ],
)


def _gru_block(z_ref, x_ref, h_ref, wxz, whz, wxr, whr, wxh, whh,
               bz, br, bh, out_ref):
    xi = jnp.concatenate([z_ref[...], x_ref[...]], axis=1)
    h = h_ref[...]
    zg = jax.nn.sigmoid(
        jnp.dot(xi, wxz[...], preferred_element_type=jnp.float32)
        + jnp.dot(h, whz[...], preferred_element_type=jnp.float32)
        + bz[...]
    )
    rg = jax.nn.sigmoid(
        jnp.dot(xi, wxr[...], preferred_element_type=jnp.float32)
        + jnp.dot(h, whr[...], preferred_element_type=jnp.float32)
        + br[...]
    )
    ht = jnp.tanh(
        jnp.dot(xi, wxh[...], preferred_element_type=jnp.float32)
        + jnp.dot(h * rg, whh[...], preferred_element_type=jnp.float32)
        + bh[...]
    )
    out_ref[...] = zg * h + (1.0 - zg) * ht


_GRU_R = 1000


def _gru(z, x, h, wxz, whz, wxr, whr, wxh, whh, bz, br, bh):
    n = z.shape[0]
    grid = (n // _GRU_R,)
    row_spec = lambda d: pl.BlockSpec((_GRU_R, d), lambda i: (i, 0))
    w_spec = pl.BlockSpec((_DH, _DH), lambda i: (0, 0))
    b_spec = pl.BlockSpec((1, _DH), lambda i: (0, 0))
    return pl.pallas_call(
        _gru_block,
        grid=grid,
        in_specs=[
            row_spec(_DZ), row_spec(_DZ), row_spec(_DH),
            w_spec, w_spec, w_spec, w_spec, w_spec, w_spec,
            b_spec, b_spec, b_spec,
        ],
        out_specs=row_spec(_DH),
        out_shape=jax.ShapeDtypeStruct((n, _DH), jnp.float32),
        compiler_params=pltpu.CompilerParams(
            dimension_semantics=("parallel",),
        ),
    )(z, x, h, wxz, whz, wxr, whr, wxh, whh, bz, br, bh)


def _fuse_block(p_ref, u_ref, wg_ref, bg_ref, out_ref):
    ge = p_ref[0] + p_ref[1]
    gl = jax.nn.relu(
        jnp.dot(u_ref[...], wg_ref[...], preferred_element_type=jnp.float32)
        + bg_ref[...]
    )
    out_ref[...] = jnp.concatenate([ge, gl], axis=1)


def _fuse(partial, u, wg, bg):
    du = u.shape[1]
    return pl.pallas_call(
        _fuse_block,
        out_shape=jax.ShapeDtypeStruct((_B, 2 * _DZ), jnp.float32),
    )(partial, u, wg, bg)


def kernel(z, u, x, edge_index, batch, batch_size, prev_h,
           W_xz, b_xz, W_hz, b_hz, W_xr, b_xr, W_hr, b_hr,
           W_xh, b_xh, W_hh, b_hh, Wg, bg):
    bidx = jnp.pad(batch.astype(jnp.int32), (0, _NW * _RPW - _N)).reshape(
        _NW, _NCHUNKS, _CHUNK
    )
    partial = _seg_sum(z, bidx)
    bz = (b_xz + b_hz).reshape(1, _DH)
    br = (b_xr + b_hr).reshape(1, _DH)
    bh = (b_xh + b_hh).reshape(1, _DH)
    H = _gru(z, x, prev_h, W_xz, W_hz, W_xr, W_hr, W_xh, W_hh, bz, br, bh)
    fused = _fuse(partial, u, Wg, bg.reshape(1, _DZ))
    return fused, H
